# baseline (speedup 1.0000x reference)
import functools

import jax
import jax.numpy as jnp
from jax import lax
from jax.experimental import pallas as pl
from jax.experimental.pallas import tpu as pltpu
from jax.experimental.pallas import tpu_sc as plsc

N = 10000
D = 128
E = 320000
A = 16
NC = 2
NS = 16
NW = NC * NS
EPW = E // NW
C = 80
M = EPW // C
NI = 4
NR = 4
K = (M - NI - 3) // NI
TAIL_LO = NI + NI * K
RPT = 624
RLAST = N - 15 * RPT
DEGN = 10240
DPT = DEGN // NS

_mesh = plsc.VectorSubcoreMesh(
    core_axis_name="c", subcore_axis_name="s", num_cores=NC, num_subcores=NS)


@functools.partial(
    pl.kernel,
    out_type=jax.ShapeDtypeStruct((NC, DEGN), jnp.float32),
    mesh=_mesh,
    scratch_types=[
        [pltpu.VMEM((C,), jnp.int32)] * NI,
        pltpu.VMEM((C,), jnp.float32),
        pltpu.VMEM((DPT,), jnp.float32),
        pltpu.VMEM_SHARED((DEGN,), jnp.float32),
        [pltpu.SemaphoreType.DMA] * NI,
        [pltpu.SemaphoreType.DMA] * NI,
    ],
)
def _sc_degree(dst_hbm, ones_hbm, deg_hbm, dstv, onesv, zv, acc,
               isem, ssem):
    cid = lax.axis_index("c")
    sid = lax.axis_index("s")
    wid = sid * NC + cid
    base = wid * EPW
    pltpu.sync_copy(ones_hbm.at[pl.ds(C, DPT)], zv)
    pltpu.sync_copy(zv, acc.at[pl.ds(sid * DPT, DPT)])
    pltpu.sync_copy(ones_hbm.at[pl.ds(0, C)], onesv)
    plsc.subcore_barrier()

    def start_idx(i, p):
        pltpu.async_copy(dst_hbm.at[pl.ds(base + i * C, C)], dstv[p], isem[p])

    def wait_idx(p):
        pltpu.make_async_copy(dst_hbm.at[pl.ds(0, C)], dstv[p], isem[p]).wait()

    def start_scat(p):
        pltpu.async_copy(onesv, acc.at[dstv[p]], ssem[p], add=True)

    def wait_scat(p):
        pltpu.make_async_copy(onesv, acc.at[dstv[p]], ssem[p]).wait()

    def step(i, p, first=False, do_sidx=True):
        if not first:
            wait_scat((p + NI - 1) % NI)
        wait_idx(p)
        start_scat(p)
        if do_sidx:
            start_idx(i + NI - 1, (p + NI - 1) % NI)

    for p in range(NI - 1):
        start_idx(p, p)
    step(0, 0, first=True)
    for i in range(1, NI):
        step(i, i)

    def body(j, carry):
        for b in range(NI):
            step(NI + j * NI + b, b)
        return carry

    lax.fori_loop(0, K, body, 0)
    for i in range(TAIL_LO, M):
        step(i, i % NI, do_sidx=(i + NI - 1 < M))
    wait_scat((M - 1) % NI)
    plsc.subcore_barrier()
    pltpu.sync_copy(acc.at[pl.ds(sid * DPT, DPT)],
                    deg_hbm.at[cid, pl.ds(sid * DPT, DPT)])


@functools.partial(
    pl.kernel,
    out_type=jax.ShapeDtypeStruct((NC, N, D), jnp.float32),
    mesh=_mesh,
    scratch_types=[
        [pltpu.VMEM((C,), jnp.int32)] * NI,
        [pltpu.VMEM((C,), jnp.int32)] * NI,
        [pltpu.VMEM((C, D), jnp.float32)] * NR,
        pltpu.VMEM_SHARED((N, D), jnp.float32),
        [pltpu.SemaphoreType.DMA] * NI,
        [pltpu.SemaphoreType.DMA] * NR,
        [pltpu.SemaphoreType.DMA] * NI,
    ],
)
def _sc_scatter(g_hbm, zeros_hbm, src_hbm, dst_hbm, out_hbm,
                srcv, dstv, rows, acc, isem, gsem, ssem):
    cid = lax.axis_index("c")
    sid = lax.axis_index("s")
    wid = sid * NC + cid

    def _rowcopy(mk_src, mk_dst):
        @pl.when(sid < NS - 1)
        def _():
            sl = pl.ds(sid * RPT, RPT)
            pltpu.sync_copy(mk_src(sl), mk_dst(sl))

        @pl.when(sid == NS - 1)
        def _():
            sl = pl.ds(15 * RPT, RLAST)
            pltpu.sync_copy(mk_src(sl), mk_dst(sl))

    @pl.when(cid == 0)
    def _():
        _rowcopy(lambda sl: g_hbm.at[sl], lambda sl: acc.at[sl])

    @pl.when(cid != 0)
    def _():
        _rowcopy(lambda sl: zeros_hbm.at[sl], lambda sl: acc.at[sl])

    plsc.subcore_barrier()
    base = wid * EPW

    def start_idx(i, p):
        pltpu.async_copy(src_hbm.at[pl.ds(base + i * C, C)], srcv[p], isem[p])
        pltpu.async_copy(dst_hbm.at[pl.ds(base + i * C, C)], dstv[p], isem[p])

    def wait_idx(p):
        pltpu.make_async_copy(src_hbm.at[pl.ds(0, C)], srcv[p], isem[p]).wait()
        pltpu.make_async_copy(dst_hbm.at[pl.ds(0, C)], dstv[p], isem[p]).wait()

    def start_gather(p4, p2):
        pltpu.async_copy(g_hbm.at[srcv[p4]], rows[p2], gsem[p2])

    def wait_gather(p4, p2):
        pltpu.make_async_copy(g_hbm.at[srcv[p4]], rows[p2], gsem[p2]).wait()

    def start_scat(p4, p2):
        pltpu.async_copy(rows[p2], acc.at[dstv[p4]], ssem[p4], add=True)

    def wait_scat(p4, p2):
        pltpu.make_async_copy(rows[p2], acc.at[dstv[p4]], ssem[p4]).wait()

    def step(i, p, first=False, do_sidx=True, do_sg2=True):
        if do_sg2:
            wait_idx((p + 2) % NI)
        if not first:
            wait_scat((p + NI - 1) % NI, (p + NI - 1) % NR)
        if do_sidx:
            start_idx(i + NI - 1, (p + NI - 1) % NI)
        if do_sg2:
            start_gather((p + 2) % NI, (p + 2) % NR)
        wait_gather(p, p)
        start_scat(p, p)

    for p in range(NI - 1):
        start_idx(p, p)
    wait_idx(0)
    start_gather(0, 0)
    wait_idx(1)
    start_gather(1, 1)
    step(0, 0, first=True)
    for i in range(1, NI):
        step(i, i % NI)

    def body(j, carry):
        for b in range(NI):
            i = NI + j * NI + b
            step(i, b)
        return carry

    lax.fori_loop(0, K, body, 0)
    for i in range(TAIL_LO, M):
        step(i, i % NI,
             do_sidx=(i + NI - 1 < M), do_sg2=(i + 2 < M))
    wait_scat((M - 1) % NI, (M - 1) % NR)
    plsc.subcore_barrier()
    _rowcopy(lambda sl: acc.at[sl], lambda sl: out_hbm.at[cid, sl])


B = 1000
GRID = N // B


def _tc_g1_body(x_ref, w1_ref, d0_ref, d1_ref, g1_ref):
    dis = lax.rsqrt(1.0 + d0_ref[...] + d1_ref[...])
    h = jnp.dot(x_ref[...], w1_ref[...],
                preferred_element_type=jnp.float32,
                precision=lax.Precision.HIGHEST)
    g1_ref[...] = dis * h


def _tc_g2_body(a0_ref, a1_ref, d0_ref, d1_ref, b1_ref, w2_ref, g2_ref):
    dis = lax.rsqrt(1.0 + d0_ref[...] + d1_ref[...])
    s = a0_ref[...].astype(jnp.float32) + a1_ref[...].astype(jnp.float32)
    u = jnp.maximum(dis * s + b1_ref[...], 0.0)
    h = jnp.dot(u, w2_ref[...],
                preferred_element_type=jnp.float32,
                precision=lax.Precision.HIGHEST)
    g2_ref[...] = dis * h


def _tc_head_body(a0_ref, a1_ref, d0_ref, d1_ref, b2_ref, lw_ref, lb_ref,
                  out_ref, sacc):
    i = pl.program_id(0)

    @pl.when(i == 0)
    def _():
        sacc[...] = jnp.zeros_like(sacc)

    dis = lax.rsqrt(1.0 + d0_ref[...] + d1_ref[...])
    s = a0_ref[...].astype(jnp.float32) + a1_ref[...].astype(jnp.float32)
    u = jnp.maximum(dis * s + b2_ref[...], 0.0)
    sacc[...] += jnp.sum(u, axis=0, keepdims=True)

    @pl.when(i == GRID - 1)
    def _():
        pooled = sacc[...] * (1.0 / N)
        out_ref[...] = jnp.dot(pooled, lw_ref[...],
                               preferred_element_type=jnp.float32,
                               precision=lax.Precision.HIGHEST) + lb_ref[...]


_row_spec = pl.BlockSpec((B, D), lambda i: (i, 0))
_col_spec = pl.BlockSpec((B, 1), lambda i: (i, 0))
_full = lambda r, c: pl.BlockSpec((r, c), lambda i: (0, 0))

_g1_call = pl.pallas_call(
    _tc_g1_body,
    grid=(GRID,),
    in_specs=[_row_spec, _full(D, D), _col_spec, _col_spec],
    out_specs=_row_spec,
    out_shape=jax.ShapeDtypeStruct((N, D), jnp.float32),
)

_g2_call = pl.pallas_call(
    _tc_g2_body,
    grid=(GRID,),
    in_specs=[_row_spec, _row_spec, _col_spec, _col_spec,
              _full(1, D), _full(D, D)],
    out_specs=_row_spec,
    out_shape=jax.ShapeDtypeStruct((N, D), jnp.float32),
)

_head_call = pl.pallas_call(
    _tc_head_body,
    grid=(GRID,),
    in_specs=[_row_spec, _row_spec, _col_spec, _col_spec,
              _full(1, D), _full(D, A), _full(1, A)],
    out_specs=pl.BlockSpec((1, A), lambda i: (0, 0)),
    out_shape=jax.ShapeDtypeStruct((1, A), jnp.float32),
    scratch_shapes=[pltpu.VMEM((1, D), jnp.float32)],
)


def kernel(x, edge_index, W1, b1, W2, b2, lin_W, lin_b):
    ei = edge_index.astype(jnp.int32)
    srcs = ei[0]
    dsts = ei[1]
    zeros2d = jnp.zeros((N, D), jnp.float32)
    ones1d = jnp.concatenate(
        [jnp.ones((C,), jnp.float32), jnp.zeros((DPT,), jnp.float32)])

    degs = _sc_degree(dsts, ones1d)
    d0 = degs[0, :N].reshape(N, 1)
    d1 = degs[1, :N].reshape(N, 1)

    g1 = _g1_call(x, W1, d0, d1)
    accs1 = _sc_scatter(g1, zeros2d, srcs, dsts)
    g2 = _g2_call(accs1[0], accs1[1], d0, d1,
                  b1.reshape(1, D), W2)
    accs2 = _sc_scatter(g2, zeros2d, srcs, dsts)
    out = _head_call(accs2[0], accs2[1], d0, d1,
                     b2.reshape(1, D), lin_W, lin_b.reshape(1, A))
    return out


# speedup vs baseline: 1.0151x; 1.0151x over previous
import functools

import jax
import jax.numpy as jnp
from jax import lax
from jax.experimental import pallas as pl
from jax.experimental.pallas import tpu as pltpu
from jax.experimental.pallas import tpu_sc as plsc

N = 10000
D = 128
E = 320000
A = 16
NC = 2
NS = 16
NW = NC * NS
EPW = E // NW
C = 80
M = EPW // C
NI = 4
NR = 4
K = (M - NI - 3) // NI
TAIL_LO = NI + NI * K
ND = 6
DHEAD = ND
DK = (M - DHEAD - (ND - 2) - 1) // ND
DTAIL_LO = DHEAD + ND * DK
RPT = 624
RLAST = N - 15 * RPT
DEGN = 10240
DPT = DEGN // NS

_mesh = plsc.VectorSubcoreMesh(
    core_axis_name="c", subcore_axis_name="s", num_cores=NC, num_subcores=NS)


@functools.partial(
    pl.kernel,
    out_type=jax.ShapeDtypeStruct((NC, DEGN), jnp.float32),
    mesh=_mesh,
    scratch_types=[
        [pltpu.VMEM((C,), jnp.int32)] * ND,
        pltpu.VMEM((C,), jnp.float32),
        pltpu.VMEM((DPT,), jnp.float32),
        pltpu.VMEM_SHARED((DEGN,), jnp.float32),
        [pltpu.SemaphoreType.DMA] * ND,
        [pltpu.SemaphoreType.DMA] * ND,
    ],
)
def _sc_degree(dst_hbm, ones_hbm, deg_hbm, dstv, onesv, zv, acc,
               isem, ssem):
    cid = lax.axis_index("c")
    sid = lax.axis_index("s")
    wid = sid * NC + cid
    base = wid * EPW
    pltpu.sync_copy(ones_hbm.at[pl.ds(C, DPT)], zv)
    pltpu.sync_copy(zv, acc.at[pl.ds(sid * DPT, DPT)])
    pltpu.sync_copy(ones_hbm.at[pl.ds(0, C)], onesv)
    plsc.subcore_barrier()

    def start_idx(i, p):
        pltpu.async_copy(dst_hbm.at[pl.ds(base + i * C, C)], dstv[p], isem[p])

    def wait_idx(p):
        pltpu.make_async_copy(dst_hbm.at[pl.ds(0, C)], dstv[p], isem[p]).wait()

    def start_scat(p):
        pltpu.async_copy(onesv, acc.at[dstv[p]], ssem[p], add=True)

    def wait_scat(p):
        pltpu.make_async_copy(onesv, acc.at[dstv[p]], ssem[p]).wait()

    def step(i, p, lag2=True, do_sidx=True):
        if lag2:
            wait_scat((p + ND - 2) % ND)
        wait_idx(p)
        start_scat(p)
        if do_sidx:
            start_idx(i + ND - 2, (p + ND - 2) % ND)

    for p in range(ND - 2):
        start_idx(p, p)
    step(0, 0, lag2=False)
    step(1, 1, lag2=False)
    for i in range(2, DHEAD):
        step(i, i % ND)

    def body(j, carry):
        for b in range(ND):
            step(DHEAD + j * ND + b, b)
        return carry

    lax.fori_loop(0, DK, body, 0)
    for i in range(DTAIL_LO, M):
        step(i, i % ND, do_sidx=(i + ND - 2 < M))
    wait_scat((M - 2) % ND)
    wait_scat((M - 1) % ND)
    plsc.subcore_barrier()
    pltpu.sync_copy(acc.at[pl.ds(sid * DPT, DPT)],
                    deg_hbm.at[cid, pl.ds(sid * DPT, DPT)])


@functools.partial(
    pl.kernel,
    out_type=jax.ShapeDtypeStruct((NC, N, D), jnp.float32),
    mesh=_mesh,
    scratch_types=[
        [pltpu.VMEM((C,), jnp.int32)] * NI,
        [pltpu.VMEM((C,), jnp.int32)] * NI,
        [pltpu.VMEM((C, D), jnp.float32)] * NR,
        pltpu.VMEM_SHARED((N, D), jnp.float32),
        [pltpu.SemaphoreType.DMA] * NI,
        [pltpu.SemaphoreType.DMA] * NR,
        [pltpu.SemaphoreType.DMA] * NI,
    ],
)
def _sc_scatter(g_hbm, zeros_hbm, src_hbm, dst_hbm, out_hbm,
                srcv, dstv, rows, acc, isem, gsem, ssem):
    cid = lax.axis_index("c")
    sid = lax.axis_index("s")
    wid = sid * NC + cid

    def _rowcopy(mk_src, mk_dst):
        @pl.when(sid < NS - 1)
        def _():
            sl = pl.ds(sid * RPT, RPT)
            pltpu.sync_copy(mk_src(sl), mk_dst(sl))

        @pl.when(sid == NS - 1)
        def _():
            sl = pl.ds(15 * RPT, RLAST)
            pltpu.sync_copy(mk_src(sl), mk_dst(sl))

    @pl.when(cid == 0)
    def _():
        _rowcopy(lambda sl: g_hbm.at[sl], lambda sl: acc.at[sl])

    @pl.when(cid != 0)
    def _():
        _rowcopy(lambda sl: zeros_hbm.at[sl], lambda sl: acc.at[sl])

    plsc.subcore_barrier()
    base = wid * EPW

    def start_idx(i, p):
        pltpu.async_copy(src_hbm.at[pl.ds(base + i * C, C)], srcv[p], isem[p])
        pltpu.async_copy(dst_hbm.at[pl.ds(base + i * C, C)], dstv[p], isem[p])

    def wait_idx(p):
        pltpu.make_async_copy(src_hbm.at[pl.ds(0, C)], srcv[p], isem[p]).wait()
        pltpu.make_async_copy(dst_hbm.at[pl.ds(0, C)], dstv[p], isem[p]).wait()

    def start_gather(p4, p2):
        pltpu.async_copy(g_hbm.at[srcv[p4]], rows[p2], gsem[p2])

    def wait_gather(p4, p2):
        pltpu.make_async_copy(g_hbm.at[srcv[p4]], rows[p2], gsem[p2]).wait()

    def start_scat(p4, p2):
        pltpu.async_copy(rows[p2], acc.at[dstv[p4]], ssem[p4], add=True)

    def wait_scat(p4, p2):
        pltpu.make_async_copy(rows[p2], acc.at[dstv[p4]], ssem[p4]).wait()

    def step(i, p, first=False, do_sidx=True, do_sg2=True):
        if do_sg2:
            wait_idx((p + 2) % NI)
        if not first:
            wait_scat((p + NI - 1) % NI, (p + NI - 1) % NR)
        if do_sidx:
            start_idx(i + NI - 1, (p + NI - 1) % NI)
        if do_sg2:
            start_gather((p + 2) % NI, (p + 2) % NR)
        wait_gather(p, p)
        start_scat(p, p)

    for p in range(NI - 1):
        start_idx(p, p)
    wait_idx(0)
    start_gather(0, 0)
    wait_idx(1)
    start_gather(1, 1)
    step(0, 0, first=True)
    for i in range(1, NI):
        step(i, i % NI)

    def body(j, carry):
        for b in range(NI):
            i = NI + j * NI + b
            step(i, b)
        return carry

    lax.fori_loop(0, K, body, 0)
    for i in range(TAIL_LO, M):
        step(i, i % NI,
             do_sidx=(i + NI - 1 < M), do_sg2=(i + 2 < M))
    wait_scat((M - 1) % NI, (M - 1) % NR)
    plsc.subcore_barrier()
    _rowcopy(lambda sl: acc.at[sl], lambda sl: out_hbm.at[cid, sl])


B = 1000
GRID = N // B


def _tc_g1_body(x_ref, w1_ref, d0_ref, d1_ref, g1_ref):
    dis = lax.rsqrt(1.0 + d0_ref[...] + d1_ref[...])
    h = jnp.dot(x_ref[...], w1_ref[...],
                preferred_element_type=jnp.float32,
                precision=lax.Precision.HIGHEST)
    g1_ref[...] = dis * h


def _tc_g2_body(a0_ref, a1_ref, d0_ref, d1_ref, b1_ref, w2_ref, g2_ref):
    dis = lax.rsqrt(1.0 + d0_ref[...] + d1_ref[...])
    s = a0_ref[...].astype(jnp.float32) + a1_ref[...].astype(jnp.float32)
    u = jnp.maximum(dis * s + b1_ref[...], 0.0)
    h = jnp.dot(u, w2_ref[...],
                preferred_element_type=jnp.float32,
                precision=lax.Precision.HIGHEST)
    g2_ref[...] = dis * h


def _tc_head_body(a0_ref, a1_ref, d0_ref, d1_ref, b2_ref, lw_ref, lb_ref,
                  out_ref, sacc):
    i = pl.program_id(0)

    @pl.when(i == 0)
    def _():
        sacc[...] = jnp.zeros_like(sacc)

    dis = lax.rsqrt(1.0 + d0_ref[...] + d1_ref[...])
    s = a0_ref[...].astype(jnp.float32) + a1_ref[...].astype(jnp.float32)
    u = jnp.maximum(dis * s + b2_ref[...], 0.0)
    sacc[...] += jnp.sum(u, axis=0, keepdims=True)

    @pl.when(i == GRID - 1)
    def _():
        pooled = sacc[...] * (1.0 / N)
        out_ref[...] = jnp.dot(pooled, lw_ref[...],
                               preferred_element_type=jnp.float32,
                               precision=lax.Precision.HIGHEST) + lb_ref[...]


_row_spec = pl.BlockSpec((B, D), lambda i: (i, 0))
_col_spec = pl.BlockSpec((B, 1), lambda i: (i, 0))
_full = lambda r, c: pl.BlockSpec((r, c), lambda i: (0, 0))

_g1_call = pl.pallas_call(
    _tc_g1_body,
    grid=(GRID,),
    in_specs=[_row_spec, _full(D, D), _col_spec, _col_spec],
    out_specs=_row_spec,
    out_shape=jax.ShapeDtypeStruct((N, D), jnp.float32),
)

_g2_call = pl.pallas_call(
    _tc_g2_body,
    grid=(GRID,),
    in_specs=[_row_spec, _row_spec, _col_spec, _col_spec,
              _full(1, D), _full(D, D)],
    out_specs=_row_spec,
    out_shape=jax.ShapeDtypeStruct((N, D), jnp.float32),
)

_head_call = pl.pallas_call(
    _tc_head_body,
    grid=(GRID,),
    in_specs=[_row_spec, _row_spec, _col_spec, _col_spec,
              _full(1, D), _full(D, A), _full(1, A)],
    out_specs=pl.BlockSpec((1, A), lambda i: (0, 0)),
    out_shape=jax.ShapeDtypeStruct((1, A), jnp.float32),
    scratch_shapes=[pltpu.VMEM((1, D), jnp.float32)],
)


def kernel(x, edge_index, W1, b1, W2, b2, lin_W, lin_b):
    ei = edge_index.astype(jnp.int32)
    srcs = ei[0]
    dsts = ei[1]
    zeros2d = jnp.zeros((N, D), jnp.float32)
    ones1d = jnp.concatenate(
        [jnp.ones((C,), jnp.float32), jnp.zeros((DPT,), jnp.float32)])

    degs = _sc_degree(dsts, ones1d)
    d0 = degs[0, :N].reshape(N, 1)
    d1 = degs[1, :N].reshape(N, 1)

    g1 = _g1_call(x, W1, d0, d1)
    accs1 = _sc_scatter(g1, zeros2d, srcs, dsts)
    g2 = _g2_call(accs1[0], accs1[1], d0, d1,
                  b1.reshape(1, D), W2)
    accs2 = _sc_scatter(g2, zeros2d, srcs, dsts)
    out = _head_call(accs2[0], accs2[1], d0, d1,
                     b2.reshape(1, D), lin_W, lin_b.reshape(1, A))
    return out
